# Initial kernel scaffold; baseline (speedup 1.0000x reference)
import jax
import jax.numpy as jnp
from jax.experimental import pallas as pl


def kernel(x_cellline, x_drug, edge_index_sen, edge_index_rev, edge_label_index, W1_sen_msg, W1_sen_root, b1_sen, W1_rev_msg, W1_rev_root, b1_rev, W2_sen_msg, W2_sen_root, b2_sen, W2_rev_msg, W2_rev_root, b2_rev, W_lin1, b_lin1, W_lin2, b_lin2):
    raise NotImplementedError("write your pallas kernel here")


# speedup vs baseline: 2.4018x; 2.4018x over previous
import functools

import jax
import jax.numpy as jnp
from jax import lax
from jax.experimental import pallas as pl
from jax.experimental.pallas import tpu as pltpu
from jax.experimental.pallas import tpu_sc as plsc

f32 = jnp.float32
i32 = jnp.int32

N = 10000
HID = 64
TW = 2 * HID
NPAD = 10240
ROWS_PER_TILE = NPAD // 16
E = 320000
CH = 128
CHUNKS_PER_TILE = 157
EPAD = 16 * CHUNKS_PER_TILE * CH
DUMP = 10200
EL = 200000
DEC_CHUNKS = 50
ELPAD = 32 * DEC_CHUNKS * CH
NC, NS = 2, 16


def _proj_body(x_ref, w_ref, o1_ref, o2_ref):
    acc = jnp.dot(x_ref[...], w_ref[...], preferred_element_type=f32)
    o1_ref[...] = acc
    o2_ref[...] = acc[:, HID:]


def _proj(x, w):
    m, k = x.shape
    bm = 400
    return pl.pallas_call(
        _proj_body,
        grid=(m // bm,),
        in_specs=[pl.BlockSpec((bm, k), lambda i: (i, 0)),
                  pl.BlockSpec((k, TW), lambda i: (0, 0))],
        out_specs=[pl.BlockSpec((bm, TW), lambda i: (i, 0)),
                   pl.BlockSpec((bm, HID), lambda i: (i, 0))],
        out_shape=[jax.ShapeDtypeStruct((m, TW), f32),
                   jax.ShapeDtypeStruct((m, HID), f32)],
    )(x, w)


def _comb_body(mean_ref, root_ref, b_ref, w_ref, o1_ref, o2_ref):
    h = jnp.maximum(mean_ref[...][:, :HID] + root_ref[...] + b_ref[...], 0.0)
    acc = jnp.dot(h, w_ref[...], preferred_element_type=f32)
    o1_ref[...] = acc
    o2_ref[...] = acc[:, HID:]


def _comb(mean, root, b_row, wcat):
    bm = 400
    return pl.pallas_call(
        _comb_body,
        grid=(N // bm,),
        in_specs=[pl.BlockSpec((bm, TW), lambda i: (i, 0)),
                  pl.BlockSpec((bm, HID), lambda i: (i, 0)),
                  pl.BlockSpec((1, HID), lambda i: (0, 0)),
                  pl.BlockSpec((HID, TW), lambda i: (0, 0))],
        out_specs=[pl.BlockSpec((bm, TW), lambda i: (i, 0)),
                   pl.BlockSpec((bm, HID), lambda i: (i, 0))],
        out_shape=[jax.ShapeDtypeStruct((N, TW), f32),
                   jax.ShapeDtypeStruct((N, HID), f32)],
    )(mean, root, b_row, wcat)


def _decproj_body(mc_ref, rc_ref, bc_ref, wc_ref, bl_ref,
                  md_ref, rd_ref, bd_ref, wd_ref, o_ref):
    z_cell = mc_ref[...][:, :HID] + rc_ref[...] + bc_ref[...]
    p_cell = jnp.dot(z_cell, wc_ref[...], preferred_element_type=f32) + bl_ref[...]
    z_drug = md_ref[...][:, :HID] + rd_ref[...] + bd_ref[...]
    p_drug = jnp.dot(z_drug, wd_ref[...], preferred_element_type=f32)
    o_ref[...] = jnp.concatenate([p_cell, p_drug], axis=1)


def _decproj(mean_c, root_c, b_c, w_c, b_l, mean_d, root_d, b_d, w_d):
    bm = 400
    row = lambda i: (i, 0)
    fixed = lambda i: (0, 0)
    return pl.pallas_call(
        _decproj_body,
        grid=(N // bm,),
        in_specs=[pl.BlockSpec((bm, TW), row),
                  pl.BlockSpec((bm, HID), row),
                  pl.BlockSpec((1, HID), fixed),
                  pl.BlockSpec((HID, HID), fixed),
                  pl.BlockSpec((1, HID), fixed),
                  pl.BlockSpec((bm, TW), row),
                  pl.BlockSpec((bm, HID), row),
                  pl.BlockSpec((1, HID), fixed),
                  pl.BlockSpec((HID, HID), fixed)],
        out_specs=pl.BlockSpec((bm, TW), row),
        out_shape=jax.ShapeDtypeStruct((N, TW), f32),
    )(mean_c, root_c, b_c, w_c, b_l, mean_d, root_d, b_d, w_d)


def _fill_zeros_2d(ref, nrows, ncols):
    z = jnp.zeros((16,), f32)

    @pl.loop(0, nrows, unroll=4)
    def _(r):
        for q in range(ncols // 16):
            ref[r, pl.ds(q * 16, 16)] = z


def _fill_const_1d(ref, n, val):
    v = jnp.full((16,), val, f32)

    @pl.loop(0, n // 16, unroll=4)
    def _(k):
        ref[pl.ds(k * 16, 16)] = v


def _make_sc_conv(compute_deg):
    mesh = plsc.VectorSubcoreMesh(core_axis_name="c", subcore_axis_name="s",
                                  num_cores=NC, num_subcores=NS)
    out_type = [jax.ShapeDtypeStruct((NPAD, TW), f32),
                jax.ShapeDtypeStruct((NPAD, TW), f32)]
    if compute_deg:
        out_type += [jax.ShapeDtypeStruct((NPAD,), f32),
                     jax.ShapeDtypeStruct((NPAD,), f32)]
    scratch = [
        pltpu.VMEM((CH,), i32),
        pltpu.VMEM((CH,), i32),
        pltpu.VMEM((CH, TW), f32),
        pltpu.VMEM((CH,), f32),
        pltpu.VMEM((ROWS_PER_TILE,), f32),
        pltpu.VMEM_SHARED((NPAD, TW), f32),
        pltpu.VMEM_SHARED((NPAD,), f32),
        pltpu.SemaphoreType.DMA,
    ]

    def body(*refs):
        if compute_deg:
            (msg_sen, msg_rev, src_sen, dst_sen, src_rev, dst_rev,
             out_sen, out_rev, dego_sen, dego_rev,
             idx_s, idx_d, rows, ones, degv, acc_sh, deg_sh, sem) = refs
            deg_sen_in = deg_rev_in = None
        else:
            (msg_sen, msg_rev, src_sen, dst_sen, src_rev, dst_rev,
             deg_sen_in, deg_rev_in,
             out_sen, out_rev,
             idx_s, idx_d, rows, ones, degv, acc_sh, deg_sh, sem) = refs
            dego_sen = dego_rev = None

        c = lax.axis_index("c")
        s = lax.axis_index("s")

        _fill_zeros_2d(rows, CH, TW)
        for k in range(ROWS_PER_TILE // CH):
            pltpu.sync_copy(rows, acc_sh.at[pl.ds(s * ROWS_PER_TILE + k * CH, CH)])
        if compute_deg:
            _fill_const_1d(degv, ROWS_PER_TILE, 0.0)
            pltpu.sync_copy(degv, deg_sh.at[pl.ds(s * ROWS_PER_TILE, ROWS_PER_TILE)])
            _fill_const_1d(ones, CH, 1.0)
        plsc.subcore_barrier()

        def run_edges(msg_hbm, src_hbm, dst_hbm):
            @pl.loop(0, CHUNKS_PER_TILE)
            def _(j):
                base = s * (CHUNKS_PER_TILE * CH) + j * CH
                pltpu.sync_copy(src_hbm.at[pl.ds(base, CH)], idx_s)
                pltpu.sync_copy(dst_hbm.at[pl.ds(base, CH)], idx_d)
                pltpu.async_copy(msg_hbm.at[idx_s], rows, sem).wait()
                pltpu.sync_copy(rows, acc_sh.at[idx_d], add=True)
                if compute_deg:
                    pltpu.sync_copy(ones, deg_sh.at[idx_d], add=True)

        @pl.when(c == 0)
        def _():
            run_edges(msg_sen, src_sen, dst_sen)

        @pl.when(c == 1)
        def _():
            run_edges(msg_rev, src_rev, dst_rev)

        plsc.subcore_barrier()

        def writeback(out_hbm, dego_hbm, deg_in_hbm):
            rbase0 = s * ROWS_PER_TILE
            if compute_deg:
                pltpu.sync_copy(deg_sh.at[pl.ds(rbase0, ROWS_PER_TILE)], degv)
                pltpu.sync_copy(degv, dego_hbm.at[pl.ds(rbase0, ROWS_PER_TILE)])
            else:
                pltpu.sync_copy(deg_in_hbm.at[pl.ds(rbase0, ROWS_PER_TILE)], degv)

            @pl.loop(0, ROWS_PER_TILE // 16, unroll=4)
            def _(k):
                d = degv[pl.ds(k * 16, 16)]
                degv[pl.ds(k * 16, 16)] = 1.0 / jnp.maximum(d, 1.0)

            for k in range(ROWS_PER_TILE // CH):
                rbase = rbase0 + k * CH
                pltpu.sync_copy(acc_sh.at[pl.ds(rbase, CH)], rows)

                @pl.loop(0, CH // 16)
                def _(g):
                    dv = degv[pl.ds(k * CH + g * 16, 16)]
                    for r16 in range(16):
                        iv = dv[r16]
                        r = g * 16 + r16
                        for q in range(HID // 16):
                            rows[r, pl.ds(q * 16, 16)] = rows[r, pl.ds(q * 16, 16)] * iv

                pltpu.sync_copy(rows, out_hbm.at[pl.ds(rbase, CH)])

        @pl.when(c == 0)
        def _():
            writeback(out_sen, dego_sen, deg_sen_in)

        @pl.when(c == 1)
        def _():
            writeback(out_rev, dego_rev, deg_rev_in)

    return functools.partial(pl.kernel, mesh=mesh, out_type=out_type,
                             scratch_types=scratch)(body)


_sc_conv_deg = _make_sc_conv(True)
_sc_conv_nodeg = _make_sc_conv(False)


def _dec_body(ptab, rowh, colh, w2b, out,
              idx_r, idx_c, pc, pd, wv, res, sem1, sem2):
    c = lax.axis_index("c")
    s = lax.axis_index("s")
    wid = s * NC + c

    pltpu.sync_copy(w2b, wv)
    w = [wv[pl.ds(q * 16, 16)] for q in range(HID // 16)]

    @pl.loop(0, DEC_CHUNKS)
    def _(j):
        base = wid * (DEC_CHUNKS * CH) + j * CH
        pltpu.sync_copy(rowh.at[pl.ds(base, CH)], idx_r)
        pltpu.sync_copy(colh.at[pl.ds(base, CH)], idx_c)
        cp1 = pltpu.async_copy(ptab.at[idx_r], pc, sem1)
        cp2 = pltpu.async_copy(ptab.at[idx_c], pd, sem2)
        cp1.wait()
        cp2.wait()

        @pl.loop(0, CH, unroll=4)
        def _(e):
            t = jnp.maximum(pc[e, pl.ds(0, 16)] + pd[e, pl.ds(HID, 16)], 0.0) * w[0]
            for q in range(1, HID // 16):
                t = t + jnp.maximum(pc[e, pl.ds(q * 16, 16)]
                                    + pd[e, pl.ds(HID + q * 16, 16)], 0.0) * w[q]
            res[e, pl.ds(0, 16)] = t

        pltpu.sync_copy(res, out.at[pl.ds(base, CH)])


_sc_dec = functools.partial(
    pl.kernel,
    mesh=plsc.VectorSubcoreMesh(core_axis_name="c", subcore_axis_name="s",
                                num_cores=NC, num_subcores=NS),
    out_type=jax.ShapeDtypeStruct((ELPAD, 16), f32),
    scratch_types=[
        pltpu.VMEM((CH,), i32),
        pltpu.VMEM((CH,), i32),
        pltpu.VMEM((CH, TW), f32),
        pltpu.VMEM((CH, TW), f32),
        pltpu.VMEM((80,), f32),
        pltpu.VMEM((CH, 16), f32),
        pltpu.SemaphoreType.DMA,
        pltpu.SemaphoreType.DMA,
    ],
)(_dec_body)


def _dec_reduce_body(t_ref, b_ref, o_ref):
    s = jnp.sum(t_ref[...], axis=1) + b_ref[0, 0]
    o_ref[...] = s.reshape(16, 128)


def _dec_reduce(tbuf, b2):
    bm = 2048
    return pl.pallas_call(
        _dec_reduce_body,
        grid=(ELPAD // bm,),
        in_specs=[pl.BlockSpec((bm, 16), lambda i: (i, 0)),
                  pl.BlockSpec((1, 1), lambda i: (0, 0))],
        out_specs=pl.BlockSpec((16, 128), lambda i: (i, 0)),
        out_shape=jax.ShapeDtypeStruct((ELPAD // 128, 128), f32),
    )(tbuf, b2)


def _pad_edges(edge_index):
    pad = EPAD - E
    src = jnp.concatenate([edge_index[0].astype(i32), jnp.zeros((pad,), i32)])
    dst = jnp.concatenate([edge_index[1].astype(i32), jnp.full((pad,), DUMP, i32)])
    return src, dst


def kernel(x_cellline, x_drug, edge_index_sen, edge_index_rev, edge_label_index,
           W1_sen_msg, W1_sen_root, b1_sen, W1_rev_msg, W1_rev_root, b1_rev,
           W2_sen_msg, W2_sen_root, b2_sen, W2_rev_msg, W2_rev_root, b2_rev,
           W_lin1, b_lin1, W_lin2, b_lin2):
    wc1 = jnp.concatenate([W1_sen_msg, W1_rev_root], axis=1)
    wd1 = jnp.concatenate([W1_rev_msg, W1_sen_root], axis=1)
    msg_sen_t, root_cell = _proj(x_cellline, wc1)
    msg_rev_t, root_drug = _proj(x_drug, wd1)

    src_sen, dst_sen = _pad_edges(edge_index_sen)
    src_rev, dst_rev = _pad_edges(edge_index_rev)

    mean_sen, mean_rev, deg_sen, deg_rev = _sc_conv_deg(
        msg_sen_t, msg_rev_t, src_sen, dst_sen, src_rev, dst_rev)

    w2c = jnp.concatenate([W2_sen_msg, W2_rev_root], axis=1)
    w2d = jnp.concatenate([W2_rev_msg, W2_sen_root], axis=1)
    m2_sen_t, root2_cell = _comb(mean_rev, root_cell, b1_rev.reshape(1, HID), w2c)
    m2_rev_t, root2_drug = _comb(mean_sen, root_drug, b1_sen.reshape(1, HID), w2d)

    mean2_sen, mean2_rev = _sc_conv_nodeg(
        m2_sen_t, m2_rev_t, src_sen, dst_sen, src_rev, dst_rev, deg_sen, deg_rev)

    ptab = _decproj(mean2_rev, root2_cell, b2_rev.reshape(1, HID),
                    W_lin1[:HID], b_lin1.reshape(1, HID),
                    mean2_sen, root2_drug, b2_sen.reshape(1, HID),
                    W_lin1[HID:])

    pad = ELPAD - EL
    rowp = jnp.concatenate([edge_label_index[0].astype(i32), jnp.zeros((pad,), i32)])
    colp = jnp.concatenate([edge_label_index[1].astype(i32), jnp.zeros((pad,), i32)])
    w2b = jnp.concatenate([W_lin2[:, 0], jnp.zeros((16,), f32)])
    tbuf = _sc_dec(ptab, rowp, colp, w2b)
    out = _dec_reduce(tbuf, b_lin2.reshape(1, 1))
    return out.reshape(-1)[:EL]
